# baseline (speedup 1.0000x reference)
import functools

import jax
import jax.numpy as jnp
from jax import lax
from jax.experimental import pallas as pl
from jax.experimental.pallas import tpu as pltpu
from jax.experimental.pallas import tpu_sc as plsc

_B = 16384
_L = 16
_NC = 2
_NS = 16
_NW = _NC * _NS
_BW = _B // _NW
_N = 1000000
_NPAD = 1000448
_G = 4 * _BW


def _sc_body(uid_hbm, iid_hbm, mu_hbm, up_hbm, tab_hbm, gab_hbm, gbb_hbm,
             mu_out, up_out, alpha_hbm, beta_hbm,
             idx_v, g_v, mu_v, up_v, a_v, b_v, gab_v, gbb_v, sem, sem2):
    wid = lax.axis_index("s") * _NC + lax.axis_index("c")
    base = wid * _BW
    sl_w = pl.ds(base, _BW)

    pltpu.sync_copy(uid_hbm.at[sl_w], idx_v.at[pl.ds(0, _BW)])
    pltpu.sync_copy(iid_hbm.at[sl_w], idx_v.at[pl.ds(_BW, _BW)])
    pltpu.sync_copy(uid_hbm.at[sl_w], idx_v.at[pl.ds(2 * _BW, _BW)])
    pltpu.sync_copy(iid_hbm.at[sl_w], idx_v.at[pl.ds(3 * _BW, _BW)])
    for seg in (1, 2, 3):
        off = jnp.int32(seg * _NPAD)
        for i in range(_BW // _L):
            sl = pl.ds(seg * _BW + i * _L, _L)
            idx_v[sl] = idx_v[sl] + off
    cg = pltpu.async_copy(tab_hbm.at[idx_v], g_v, sem)
    m1 = pltpu.async_copy(mu_hbm.at[sl_w], mu_out.at[sl_w], sem2)
    m2 = pltpu.async_copy(up_hbm.at[sl_w], up_out.at[sl_w], sem2)
    pltpu.sync_copy(mu_hbm.at[sl_w], mu_v)
    pltpu.sync_copy(up_hbm.at[sl_w], up_v)
    pltpu.sync_copy(gab_hbm, gab_v)
    pltpu.sync_copy(gbb_hbm, gbb_v)

    eps = jnp.float32(0.01)
    ga = gab_v[...]
    gb = gbb_v[...]
    cg.wait()
    for i in range(_BW // _L):
        sl = pl.ds(i * _L, _L)
        mu16 = mu_v[sl]
        up16 = up_v[sl]
        raw = jnp.maximum(mu16 * up16, eps)
        ua16 = g_v[pl.ds(i * _L, _L)]
        ia16 = g_v[pl.ds(_BW + i * _L, _L)]
        ub16 = g_v[pl.ds(2 * _BW + i * _L, _L)]
        ib16 = g_v[pl.ds(3 * _BW + i * _L, _L)]
        a_v[sl] = jnp.maximum(raw + ga + ua16 + ia16, eps)
        b_v[sl] = jnp.maximum(jnp.maximum(up16 - raw, eps) + gb + ub16 + ib16,
                              eps)
    pltpu.sync_copy(a_v, alpha_hbm.at[sl_w])
    pltpu.sync_copy(b_v, beta_hbm.at[sl_w])

    m1.wait()
    m2.wait()


_sc_call = pl.kernel(
    _sc_body,
    out_type=(jax.ShapeDtypeStruct((_B,), jnp.float32),
              jax.ShapeDtypeStruct((_B,), jnp.float32),
              jax.ShapeDtypeStruct((_B,), jnp.float32),
              jax.ShapeDtypeStruct((_B,), jnp.float32)),
    mesh=plsc.VectorSubcoreMesh(core_axis_name="c", subcore_axis_name="s"),
    scratch_types=[
        pltpu.VMEM((_G,), jnp.int32),
        pltpu.VMEM((_G,), jnp.float32),
        pltpu.VMEM((_BW,), jnp.float32),
        pltpu.VMEM((_BW,), jnp.float32),
        pltpu.VMEM((_BW,), jnp.float32),
        pltpu.VMEM((_BW,), jnp.float32),
        pltpu.VMEM((_L,), jnp.float32),
        pltpu.VMEM((_L,), jnp.float32),
        pltpu.SemaphoreType.DMA,
        pltpu.SemaphoreType.DMA,
    ],
)


@jax.jit
def kernel(uid, iid, mu, upsilon, uid_alpha_emb, iid_alpha_emb,
           uid_beta_emb, iid_beta_emb, g_alpha_bias, g_beta_bias):
    ga = jnp.full((_L,), g_alpha_bias, jnp.float32)
    gb = jnp.full((_L,), g_beta_bias, jnp.float32)
    zpad = jnp.zeros((_NPAD - _N, 1), jnp.float32)
    tab = jnp.concatenate(
        [uid_alpha_emb, zpad, iid_alpha_emb, zpad,
         uid_beta_emb, zpad, iid_beta_emb, zpad], axis=0).reshape(-1)
    mu_o, up_o, alpha, beta = _sc_call(
        uid.astype(jnp.int32), iid.astype(jnp.int32),
        mu.reshape(-1), upsilon.reshape(-1), tab, ga, gb)
    return (mu_o.reshape(-1, 1), up_o.reshape(-1, 1),
            alpha.reshape(-1, 1), beta.reshape(-1, 1))


# speedup vs baseline: 5.6276x; 5.6276x over previous
import functools

import jax
import jax.numpy as jnp
from jax import lax
from jax.experimental import pallas as pl
from jax.experimental.pallas import tpu as pltpu
from jax.experimental.pallas import tpu_sc as plsc

_B = 16384
_L = 16
_NC = 2
_NS = 16
_NW = _NC * _NS
_BW = _B // _NW
_N = 1000000
_NPAD = 1000448


def _make_half(alpha_side):
    def body(uid_hbm, iid_hbm, mu_hbm, up_hbm, tu_hbm, ti_hbm, g_hbm,
             pass_out, res_hbm,
             uidx_v, iidx_v, mu_v, up_v, tu_v, ti_v, r_v, g_v, sem, sem2):
        wid = lax.axis_index("s") * _NC + lax.axis_index("c")
        base = wid * _BW
        sl_w = pl.ds(base, _BW)

        pltpu.sync_copy(uid_hbm.at[sl_w], uidx_v)
        pltpu.sync_copy(iid_hbm.at[sl_w], iidx_v)
        c1 = pltpu.async_copy(tu_hbm.at[uidx_v], tu_v, sem)
        c2 = pltpu.async_copy(ti_hbm.at[iidx_v], ti_v, sem)
        src_hbm = mu_hbm if alpha_side else up_hbm
        m1 = pltpu.async_copy(src_hbm.at[sl_w], pass_out.at[sl_w], sem2)
        pltpu.sync_copy(mu_hbm.at[sl_w], mu_v)
        pltpu.sync_copy(up_hbm.at[sl_w], up_v)
        pltpu.sync_copy(g_hbm, g_v)

        eps = jnp.float32(0.01)
        g = g_v[...]
        c1.wait()
        c2.wait()
        for i in range(_BW // _L):
            sl = pl.ds(i * _L, _L)
            mu16 = mu_v[sl]
            up16 = up_v[sl]
            raw = jnp.maximum(mu16 * up16, eps)
            if alpha_side:
                r = raw
            else:
                r = jnp.maximum(up16 - raw, eps)
            r_v[sl] = jnp.maximum(r + g + tu_v[sl] + ti_v[sl], eps)
        pltpu.sync_copy(r_v, res_hbm.at[sl_w])
        m1.wait()

    return pl.kernel(
        body,
        out_type=(jax.ShapeDtypeStruct((_B,), jnp.float32),
                  jax.ShapeDtypeStruct((_B,), jnp.float32)),
        mesh=plsc.VectorSubcoreMesh(core_axis_name="c", subcore_axis_name="s"),
        scratch_types=[
            pltpu.VMEM((_BW,), jnp.int32),
            pltpu.VMEM((_BW,), jnp.int32),
            pltpu.VMEM((_BW,), jnp.float32),
            pltpu.VMEM((_BW,), jnp.float32),
            pltpu.VMEM((_BW,), jnp.float32),
            pltpu.VMEM((_BW,), jnp.float32),
            pltpu.VMEM((_BW,), jnp.float32),
            pltpu.VMEM((_L,), jnp.float32),
            pltpu.SemaphoreType.DMA,
            pltpu.SemaphoreType.DMA,
        ],
        name="alpha_half" if alpha_side else "beta_half",
    )


_alpha_call = _make_half(True)
_beta_call = _make_half(False)


def _flat_table(t):
    return jnp.pad(t, ((0, _NPAD - _N), (0, 0))).reshape(-1)


@jax.jit
def kernel(uid, iid, mu, upsilon, uid_alpha_emb, iid_alpha_emb,
           uid_beta_emb, iid_beta_emb, g_alpha_bias, g_beta_bias):
    ga = jnp.full((_L,), g_alpha_bias, jnp.float32)
    gb = jnp.full((_L,), g_beta_bias, jnp.float32)
    uid32 = uid.astype(jnp.int32)
    iid32 = iid.astype(jnp.int32)
    mu1 = mu.reshape(-1)
    up1 = upsilon.reshape(-1)
    mu_o, alpha = _alpha_call(
        uid32, iid32, mu1, up1,
        _flat_table(uid_alpha_emb), _flat_table(iid_alpha_emb), ga)
    up_o, beta = _beta_call(
        uid32, iid32, mu1, up1,
        _flat_table(uid_beta_emb), _flat_table(iid_beta_emb), gb)
    return (mu_o.reshape(-1, 1), up_o.reshape(-1, 1),
            alpha.reshape(-1, 1), beta.reshape(-1, 1))
